# baseline (speedup 1.0000x reference)
import functools

import jax
import jax.numpy as jnp
from jax import lax
from jax.experimental import pallas as pl
from jax.experimental.pallas import tpu as pltpu
from jax.experimental.pallas import tpu_sc as plsc

N = 320000
F = 128
P = 10000
PP = 10240
NC = 2
NT = 16
NW = NC * NT
N1 = 163840
N2 = N - N1
CR = 80
GR = CR // 16
TPP = PP // NT
L = 16


def _g16(x, idx):
    return jnp.take_along_axis(x, idx, axis=0, mode="promise_in_bounds")


def _sc_body(rpw, i_hbm, w_hbm, out_hbm, xbuf, xbuf1, w_s, c_loc,
             shared, mbuf, res_v, sem0, sem1):
    nchunk = rpw // CR
    cid = lax.axis_index("c")
    sid = lax.axis_index("s")
    wid = sid * NC + cid
    pltpu.sync_copy(w_hbm, w_s)

    zeros16 = jnp.zeros((L,), jnp.float32)

    def zero_body(k, c):
        c_loc[pl.ds(k * L, L)] = zeros16
        return c

    lax.fori_loop(0, PP // L, zero_body, 0)


    iota = lax.broadcasted_iota(jnp.int32, (L,), 0)
    nxt = jnp.minimum(iota + 1, L - 1)
    shift_idx = [jnp.maximum(iota - d, 0) for d in (1, 2, 4, 8)]
    last = iota == (L - 1)
    zcol = jnp.zeros((L,), jnp.int32)
    lane = [jnp.full((L,), k, jnp.int32) for k in range(L)]
    wv = [w_s[pl.ds(j * L, L)] for j in range(9)]
    bias = _g16(w_s[pl.ds(144, L)], zcol)
    w127 = _g16(wv[8], zcol)
    lane128 = zcol + (F + 1 - 1)
    row0 = wid * rpw

    def _start(ck, buf, sem):
        pltpu.async_copy(i_hbm.at[pl.ds(row0 + ck * CR, CR), :], buf, sem)

    def _wait(ck, buf, sem):
        pltpu.make_async_copy(
            i_hbm.at[pl.ds(row0 + ck * CR, CR), :], buf, sem
        ).wait()

    def _consume(ck, buf):
        def group_body(g, c2):
            segf = plsc.load_gather(buf, [g * L + iota, zcol])
            s = segf.astype(jnp.int32)
            v = jnp.zeros((L,), jnp.float32)
            for t in range(L):
                row = g * L + t
                prods = [buf[row, pl.ds(L * j, L)] * wv[j]
                         for j in range(8)]
                p0 = (prods[0] + prods[1]) + (prods[2] + prods[3])
                p1 = (prods[4] + prods[5]) + (prods[6] + prods[7])
                tot = p0 + p1
                for idx in shift_idx:
                    tot = tot + _g16(tot, idx)
                v = jnp.where(iota == t, _g16(tot, lane[L - 1]), v)
            rows = g * L + iota
            x127 = plsc.load_gather(buf, [rows, lane128])
            v = jnp.maximum(v + x127 * w127 + bias, 0.0)
            for idx in shift_idx:
                sv = _g16(v, idx)
                ss = _g16(s, idx)
                v = jnp.where(ss == s, jnp.maximum(v, sv), v)
            m = (s != _g16(s, nxt)) | last
            old = plsc.load_gather(c_loc, [s], mask=m)
            plsc.store_scatter(c_loc, [s], jnp.maximum(old, v), mask=m)
            return c2

        lax.fori_loop(0, GR, group_body, 0)

    _start(0, xbuf, sem0)

    def pair_body(m, carry):
        _start(2 * m + 1, xbuf1, sem1)
        _wait(2 * m, xbuf, sem0)
        _consume(2 * m, xbuf)
        _start(2 * m + 2, xbuf, sem0)
        _wait(2 * m + 1, xbuf1, sem1)
        _consume(2 * m + 1, xbuf1)
        return carry

    lax.fori_loop(0, (nchunk - 1) // 2, pair_body, 0)
    if nchunk % 2:
        _wait(nchunk - 1, xbuf, sem0)
        _consume(nchunk - 1, xbuf)
    else:
        _start(nchunk - 1, xbuf1, sem1)
        _wait(nchunk - 2, xbuf, sem0)
        _consume(nchunk - 2, xbuf)
        _wait(nchunk - 1, xbuf1, sem1)
        _consume(nchunk - 1, xbuf1)

    pltpu.sync_copy(c_loc, shared.at[sid])
    plsc.subcore_barrier()
    pltpu.sync_copy(shared.at[:, pl.ds(sid * TPP, TPP)], mbuf)

    def merge_body(j, c):
        acc = mbuf[0, pl.ds(j * L, L)]
        for t in range(1, NT):
            acc = jnp.maximum(acc, mbuf[t, pl.ds(j * L, L)])
        res_v[pl.ds(j * L, L)] = acc
        return c

    lax.fori_loop(0, TPP // L, merge_body, 0)
    pltpu.sync_copy(res_v, out_hbm.at[cid, pl.ds(sid * TPP, TPP)])


def _make_sc(rpw):
    return functools.partial(
        pl.kernel,
        out_type=jax.ShapeDtypeStruct((NC, PP), jnp.float32),
        mesh=plsc.VectorSubcoreMesh(core_axis_name="c", subcore_axis_name="s"),
        compiler_params=pltpu.CompilerParams(
            needs_layout_passes=False, use_tc_tiling_on_sc=True),
        scratch_types=[
            pltpu.VMEM((CR, F + 1), jnp.float32),
            pltpu.VMEM((CR, F + 1), jnp.float32),
            pltpu.VMEM((160,), jnp.float32),
            pltpu.VMEM((PP,), jnp.float32),
            pltpu.VMEM_SHARED((NT, PP), jnp.float32),
            pltpu.VMEM((NT, TPP), jnp.float32),
            pltpu.VMEM((TPP,), jnp.float32),
            pltpu.SemaphoreType.DMA,
            pltpu.SemaphoreType.DMA,
        ],
    )(functools.partial(_sc_body, rpw))


_sc_segmax_1 = _make_sc(N1 // NW)
_sc_segmax_2 = _make_sc(N2 // NW)


def _merge_body(a_ref, b_ref, p_ref, o_ref):
    mx = jnp.maximum(jnp.max(a_ref[...], axis=0, keepdims=True),
                     jnp.max(b_ref[...], axis=0, keepdims=True))
    idx = lax.broadcasted_iota(jnp.int32, (1, PP), 1)
    o_ref[...] = jnp.where(idx < p_ref[0], mx, 0.0)


def _tc_merge(parts1, parts2, p):
    return pl.pallas_call(
        _merge_body,
        in_specs=[
            pl.BlockSpec(memory_space=pltpu.VMEM),
            pl.BlockSpec(memory_space=pltpu.VMEM),
            pl.BlockSpec(memory_space=pltpu.SMEM),
        ],
        out_specs=pl.BlockSpec(memory_space=pltpu.VMEM),
        out_shape=jax.ShapeDtypeStruct((1, PP), jnp.float32),
    )(parts1, parts2, p)


def kernel(_, i, p, W, b):
    w_all = jnp.concatenate([
        jnp.zeros((1,), jnp.float32), W.reshape(F),
        jnp.zeros((15,), jnp.float32), b, jnp.zeros((15,), jnp.float32),
    ])
    parts1 = _sc_segmax_1(i[:N1], w_all)
    parts2 = _sc_segmax_2(i[N1:], w_all)
    p_arr = jnp.asarray(p, jnp.int32).reshape(1)
    c = _tc_merge(parts1, parts2, p_arr)
    return c[0, :P]


# speedup vs baseline: 1.1855x; 1.1855x over previous
import functools

import jax
import jax.numpy as jnp
from jax import lax
from jax.experimental import pallas as pl
from jax.experimental.pallas import tpu as pltpu
from jax.experimental.pallas import tpu_sc as plsc

N = 320000
F = 128
P = 10000
PP = 10240
NC = 2
NT = 16
NW = NC * NT
RPW = N // NW
CR = 80
NCHUNK = RPW // CR
GR = CR // 16
TPP = PP // NT
L = 16


def _g16(x, idx):
    return jnp.take_along_axis(x, idx, axis=0, mode="promise_in_bounds")


def _sc_body(i_hbm, w_hbm, out_hbm, xbuf, xbuf1, w_s, c_loc,
             shared, mbuf, res_v, sem0, sem1):
    cid = lax.axis_index("c")
    sid = lax.axis_index("s")
    wid = sid * NC + cid
    pltpu.sync_copy(w_hbm, w_s)

    zeros16 = jnp.zeros((L,), jnp.float32)

    def zero_body(k, c):
        c_loc[pl.ds(k * L, L)] = zeros16
        return c

    lax.fori_loop(0, PP // L, zero_body, 0)


    iota = lax.broadcasted_iota(jnp.int32, (L,), 0)
    nxt = jnp.minimum(iota + 1, L - 1)
    shift_idx = [jnp.maximum(iota - d, 0) for d in (1, 2, 4, 8)]
    last = iota == (L - 1)
    zcol = jnp.zeros((L,), jnp.int32)
    lane = [jnp.full((L,), k, jnp.int32) for k in range(L)]
    wv = [w_s[pl.ds(j * L, L)] for j in range(9)]
    bias = _g16(w_s[pl.ds(144, L)], zcol)
    w127 = _g16(wv[8], zcol)
    lane128 = zcol + (F + 1 - 1)
    row0 = wid * RPW

    def _start(ck, buf, sem):
        pltpu.async_copy(i_hbm.at[pl.ds(row0 + ck * CR, CR), :], buf, sem)

    def _wait(ck, buf, sem):
        pltpu.make_async_copy(
            i_hbm.at[pl.ds(row0 + ck * CR, CR), :], buf, sem
        ).wait()

    def _consume(ck, buf):
        def group_body(g, c2):
            segf = plsc.load_gather(buf, [g * L + iota, zcol])
            s = segf.astype(jnp.int32)
            v = jnp.zeros((L,), jnp.float32)
            for t in range(L):
                row = g * L + t
                prods = [buf[row, pl.ds(L * j, L)] * wv[j]
                         for j in range(8)]
                p0 = (prods[0] + prods[1]) + (prods[2] + prods[3])
                p1 = (prods[4] + prods[5]) + (prods[6] + prods[7])
                tot = p0 + p1
                for idx in shift_idx:
                    tot = tot + _g16(tot, idx)
                v = jnp.where(iota == t, _g16(tot, lane[L - 1]), v)
            rows = g * L + iota
            x127 = plsc.load_gather(buf, [rows, lane128])
            v = jnp.maximum(v + x127 * w127 + bias, 0.0)
            for idx in shift_idx:
                sv = _g16(v, idx)
                ss = _g16(s, idx)
                v = jnp.where(ss == s, jnp.maximum(v, sv), v)
            m = (s != _g16(s, nxt)) | last
            old = plsc.load_gather(c_loc, [s], mask=m)
            plsc.store_scatter(c_loc, [s], jnp.maximum(old, v), mask=m)
            return c2

        lax.fori_loop(0, GR, group_body, 0)

    _start(0, xbuf, sem0)

    def pair_body(m, carry):
        _start(2 * m + 1, xbuf1, sem1)
        _wait(2 * m, xbuf, sem0)
        _consume(2 * m, xbuf)
        _start(2 * m + 2, xbuf, sem0)
        _wait(2 * m + 1, xbuf1, sem1)
        _consume(2 * m + 1, xbuf1)
        return carry

    lax.fori_loop(0, (NCHUNK - 1) // 2, pair_body, 0)
    _wait(NCHUNK - 1, xbuf, sem0)
    _consume(NCHUNK - 1, xbuf)

    pltpu.sync_copy(c_loc, shared.at[sid])
    plsc.subcore_barrier()
    pltpu.sync_copy(shared.at[:, pl.ds(sid * TPP, TPP)], mbuf)

    def merge_body(j, c):
        acc = mbuf[0, pl.ds(j * L, L)]
        for t in range(1, NT):
            acc = jnp.maximum(acc, mbuf[t, pl.ds(j * L, L)])
        res_v[pl.ds(j * L, L)] = acc
        return c

    lax.fori_loop(0, TPP // L, merge_body, 0)
    pltpu.sync_copy(res_v, out_hbm.at[cid, pl.ds(sid * TPP, TPP)])


_sc_segmax = functools.partial(
    pl.kernel,
    out_type=jax.ShapeDtypeStruct((NC, PP), jnp.float32),
    mesh=plsc.VectorSubcoreMesh(core_axis_name="c", subcore_axis_name="s"),
    compiler_params=pltpu.CompilerParams(
        needs_layout_passes=False, use_tc_tiling_on_sc=True),
    scratch_types=[
        pltpu.VMEM((CR, F + 1), jnp.float32),
        pltpu.VMEM((CR, F + 1), jnp.float32),
        pltpu.VMEM((160,), jnp.float32),
        pltpu.VMEM((PP,), jnp.float32),
        pltpu.VMEM_SHARED((NT, PP), jnp.float32),
        pltpu.VMEM((NT, TPP), jnp.float32),
        pltpu.VMEM((TPP,), jnp.float32),
        pltpu.SemaphoreType.DMA,
        pltpu.SemaphoreType.DMA,
    ],
)(_sc_body)


def _merge_body(a_ref, p_ref, o_ref):
    mx = jnp.max(a_ref[...], axis=0, keepdims=True)
    idx = lax.broadcasted_iota(jnp.int32, (1, PP), 1)
    o_ref[...] = jnp.where(idx < p_ref[0], mx, 0.0)


def _tc_merge(parts, p):
    return pl.pallas_call(
        _merge_body,
        in_specs=[
            pl.BlockSpec(memory_space=pltpu.VMEM),
            pl.BlockSpec(memory_space=pltpu.SMEM),
        ],
        out_specs=pl.BlockSpec(memory_space=pltpu.VMEM),
        out_shape=jax.ShapeDtypeStruct((1, PP), jnp.float32),
    )(parts, p)


def kernel(_, i, p, W, b):
    w_all = jnp.concatenate([
        jnp.zeros((1,), jnp.float32), W.reshape(F),
        jnp.zeros((15,), jnp.float32), b, jnp.zeros((15,), jnp.float32),
    ])
    parts = _sc_segmax(i, w_all)
    p_arr = jnp.asarray(p, jnp.int32).reshape(1)
    c = _tc_merge(parts, p_arr)
    return c[0, :P]
